# baseline (speedup 1.0000x reference)
import functools

import jax
import jax.numpy as jnp
from jax import lax
from jax.experimental import pallas as pl
from jax.experimental.pallas import tpu as pltpu
from jax.experimental.pallas import tpu_sc as plsc


def _make_sc_embed(B, T, V, D):
    info = plsc.get_sparse_core_info()
    NC, NS, L = info.num_cores, info.num_subcores, info.num_lanes
    NW = NC * NS

    assert T % NW == 0, (T, NW)
    P = T // NW
    C = 32
    assert P % C == 0 and D % L == 0
    NCHUNK = P // C

    mesh = plsc.VectorSubcoreMesh(core_axis_name="c", subcore_axis_name="s")

    @functools.partial(
        pl.kernel,
        mesh=mesh,
        out_type=jax.ShapeDtypeStruct((B * T, D), jnp.float32),
        scratch_types=[
            pltpu.VMEM((B * P,), jnp.int32),
            pltpu.VMEM((P, D), jnp.float32),
            pltpu.VMEM((C, D), jnp.float32),
            pltpu.SemaphoreType.DMA,
        ],
    )
    def k(idx_hbm, tok_hbm, pos_hbm, out_hbm, idx_v, pos_v, tok_v, sem):
        wid = lax.axis_index("s") * NC + lax.axis_index("c")
        pbase = wid * P

        pltpu.sync_copy(pos_hbm.at[pl.ds(pbase, P)], pos_v)
        for b in range(B):
            pltpu.sync_copy(idx_hbm.at[pl.ds(b * T + pbase, P)],
                            idx_v.at[pl.ds(b * P, P)])

        for b in range(B):
            for cc in range(NCHUNK):
                off = cc * C
                pltpu.async_copy(
                    tok_hbm.at[idx_v.at[pl.ds(b * P + off, C)]],
                    tok_v, sem).wait()

                def row_body(i, _, off=off):
                    def col_body(j, _, i=i, off=off):
                        sl = pl.ds(j * L, L)
                        tok_v[i, sl] = tok_v[i, sl] + pos_v[off + i, sl]
                        return 0
                    return lax.fori_loop(0, D // L, col_body, 0)

                lax.fori_loop(0, C, row_body, 0)
                pltpu.sync_copy(tok_v,
                                out_hbm.at[pl.ds(b * T + pbase + off, C)])

    return k


def kernel(idx, token_embedding, position_embedding):
    B, T = idx.shape
    V, D = token_embedding.shape
    k = _make_sc_embed(B, T, V, D)
    out = k(idx.reshape(B * T).astype(jnp.int32),
            token_embedding,
            position_embedding[:T])
    return out.reshape(B, T, D)


# speedup vs baseline: 1.7532x; 1.7532x over previous
import functools

import jax
import jax.numpy as jnp
from jax import lax
from jax.experimental import pallas as pl
from jax.experimental.pallas import tpu as pltpu
from jax.experimental.pallas import tpu_sc as plsc


def _make_sc_embed(B, T, V, D):
    info = plsc.get_sparse_core_info()
    NC, NS, L = info.num_cores, info.num_subcores, info.num_lanes
    NW = NC * NS

    assert T % NW == 0, (T, NW)
    P = T // NW
    C = 16
    assert P % C == 0 and D % L == 0
    NCH = B * P // C

    mesh = plsc.VectorSubcoreMesh(core_axis_name="c", subcore_axis_name="s")

    @functools.partial(
        pl.kernel,
        mesh=mesh,
        out_type=jax.ShapeDtypeStruct((B * T, D), jnp.float32),
        scratch_types=[
            pltpu.VMEM((B * P,), jnp.int32),
            pltpu.VMEM((P, D), jnp.float32),
            pltpu.VMEM((C, D), jnp.float32),
            pltpu.VMEM((C, D), jnp.float32),
            pltpu.SemaphoreType.DMA,
            pltpu.SemaphoreType.DMA,
            pltpu.SemaphoreType.DMA,
            pltpu.SemaphoreType.DMA,
        ],
    )
    def k(idx_hbm, tok_hbm, pos_hbm, out_hbm,
          idx_v, pos_v, tok_a, tok_b, gsem_a, gsem_b, osem_a, osem_b):
        wid = lax.axis_index("s") * NC + lax.axis_index("c")
        pbase = wid * P

        pltpu.sync_copy(pos_hbm.at[pl.ds(pbase, P)], pos_v)
        for b in range(B):
            pltpu.sync_copy(idx_hbm.at[pl.ds(b * T + pbase, P)],
                            idx_v.at[pl.ds(b * P, P)])

        bufs = [(tok_a, gsem_a, osem_a), (tok_b, gsem_b, osem_b)]
        gdesc = [None] * NCH
        odesc = [None] * NCH

        def start_gather(g):
            buf, gsem, _ = bufs[g % 2]
            d = pltpu.make_async_copy(
                tok_hbm.at[idx_v.at[pl.ds(g * C, C)]], buf, gsem)
            d.start()
            gdesc[g] = d

        start_gather(0)
        for g in range(NCH):
            buf, _, osem = bufs[g % 2]
            if g + 1 < NCH:
                if g >= 1:
                    odesc[g - 1].wait()
                start_gather(g + 1)
            gdesc[g].wait()

            b, off = divmod(g * C, P)

            def row_body(i, _, buf=buf, off=off):
                for j in range(D // L):
                    sl = pl.ds(j * L, L)
                    plsc.addupdate(buf.at[i, sl], pos_v[off + i, sl])
                return 0

            lax.fori_loop(0, C, row_body, 0)

            d = pltpu.make_async_copy(
                buf, out_hbm.at[pl.ds(b * T + pbase + off, C)], osem)
            d.start()
            odesc[g] = d

        odesc[NCH - 2].wait()
        odesc[NCH - 1].wait()

    return k


def kernel(idx, token_embedding, position_embedding):
    B, T = idx.shape
    V, D = token_embedding.shape
    k = _make_sc_embed(B, T, V, D)
    out = k(idx.reshape(B * T).astype(jnp.int32),
            token_embedding,
            position_embedding[:T])
    return out.reshape(B, T, D)


# speedup vs baseline: 2.1082x; 1.2025x over previous
import functools

import jax
import jax.numpy as jnp
from jax import lax
from jax.experimental import pallas as pl
from jax.experimental.pallas import tpu as pltpu
from jax.experimental.pallas import tpu_sc as plsc


def _make_sc_embed(B, T, V, D):
    info = plsc.get_sparse_core_info()
    NC, NS, L = info.num_cores, info.num_subcores, info.num_lanes
    NW = NC * NS

    assert T % NW == 0, (T, NW)
    P = T // NW
    C = 8
    HP = 32
    assert P % C == 0 and D % L == 0 and P % HP == 0 and HP % C == 0
    NG = P // C

    mesh = plsc.VectorSubcoreMesh(core_axis_name="c", subcore_axis_name="s")

    tok_scratch = [pltpu.VMEM((C, D), jnp.float32) for _ in range(2 * B)]

    @functools.partial(
        pl.kernel,
        mesh=mesh,
        out_type=jax.ShapeDtypeStruct((B * T, D), jnp.float32),
        scratch_types=[
            pltpu.VMEM((B * P,), jnp.int32),
            pltpu.VMEM((HP, D), jnp.float32),
            *tok_scratch,
            pltpu.SemaphoreType.DMA,
            pltpu.SemaphoreType.DMA,
            pltpu.SemaphoreType.DMA,
            pltpu.SemaphoreType.DMA,
        ],
    )
    def k(idx_hbm, tok_hbm, pos_hbm, out_hbm, idx_v, pos_h, *rest):
        bufs = [list(rest[:B]), list(rest[B:2 * B])]
        gsems = rest[2 * B:2 * B + 2]
        osems = rest[2 * B + 2:2 * B + 4]

        wid = lax.axis_index("s") * NC + lax.axis_index("c")
        pbase = wid * P

        for b in range(B):
            pltpu.sync_copy(idx_hbm.at[pl.ds(b * T + pbase, P)],
                            idx_v.at[pl.ds(b * P, P)])

        gdesc = [None] * NG
        odesc = [None] * NG

        def start_group(g):
            p = g % 2
            ds = []
            for b in range(B):
                d = pltpu.make_async_copy(
                    tok_hbm.at[idx_v.at[pl.ds(b * P + g * C, C)]],
                    bufs[p][b], gsems[p])
                d.start()
                ds.append(d)
            gdesc[g] = ds

        def start_out(g):
            p = g % 2
            ds = []
            for b in range(B):
                d = pltpu.make_async_copy(
                    bufs[p][b],
                    out_hbm.at[pl.ds(b * T + pbase + g * C, C)],
                    osems[p])
                d.start()
                ds.append(d)
            odesc[g] = ds

        cur_half = -1
        start_group(0)
        for g in range(NG):
            p = g % 2
            if g + 1 < NG:
                if g >= 1:
                    for d in odesc[g - 1]:
                        d.wait()
                start_group(g + 1)

            h = (g * C) // HP
            if h != cur_half:
                pltpu.sync_copy(pos_hbm.at[pl.ds(pbase + h * HP, HP)],
                                pos_h)
                cur_half = h
            off_h = g * C - h * HP

            for d in gdesc[g]:
                d.wait()

            bufs4 = bufs[p]

            def row_body(i, _, bufs4=bufs4, off_h=off_h):
                for j in range(D // L):
                    sl = pl.ds(j * L, L)
                    pv = pos_h[off_h + i, sl]
                    for b in range(B):
                        plsc.addupdate(bufs4[b].at[i, sl], pv)
                return 0

            lax.fori_loop(0, C, row_body, 0)
            start_out(g)

        for d in odesc[NG - 2]:
            d.wait()
        for d in odesc[NG - 1]:
            d.wait()

    return k


def kernel(idx, token_embedding, position_embedding):
    B, T = idx.shape
    V, D = token_embedding.shape
    k = _make_sc_embed(B, T, V, D)
    out = k(idx.reshape(B * T).astype(jnp.int32),
            token_embedding,
            position_embedding[:T])
    return out.reshape(B, T, D)
